# baseline (speedup 1.0000x reference)
import functools

import jax
import jax.numpy as jnp
from jax import lax
from jax.experimental import pallas as pl
from jax.experimental.pallas import tpu as pltpu
from jax.experimental.pallas import tpu_sc as plsc

_INFO = plsc.get_sparse_core_info()
_NC, _NS = _INFO.num_cores, _INFO.num_subcores
_NW = _NC * _NS

_NBUF = 2
_VBLK = 512


def _pack_kernel(emb_t_ref, eye_ref, out_ref):
    blk = emb_t_ref[...]
    eye = eye_ref[...]
    t = lax.dot_general(blk, eye, (((0,), (0,)), ((), ())))
    out_ref[:, 0:64] = t
    out_ref[:, 64:128] = t


@functools.lru_cache(maxsize=None)
def _make_pack(V, D):
    grid = (V + _VBLK - 1) // _VBLK
    return pl.pallas_call(
        _pack_kernel,
        grid=(grid,),
        in_specs=[
            pl.BlockSpec((D, _VBLK), lambda i: (0, i)),
            pl.BlockSpec((D, D), lambda i: (0, 0)),
        ],
        out_specs=pl.BlockSpec((_VBLK, 2 * D), lambda i: (i, 0)),
        out_shape=jax.ShapeDtypeStruct((V, 2 * D), jnp.float32),
    )


def _tile_kernel(g_ref, eye_ref, out_ref):
    blk = g_ref[0][:, 0:64]
    eye = eye_ref[...]
    t = lax.dot_general(blk, eye, (((0,), (0,)), ((), ())))
    out_ref[...] = t.reshape(1, 8, 1, 8, 128)


@functools.lru_cache(maxsize=None)
def _make_tile(B, H, D):
    nblk = B // 128
    return pl.pallas_call(
        _tile_kernel,
        grid=(H, nblk),
        in_specs=[
            pl.BlockSpec((1, 128, 2 * D), lambda h, j: (h, j, 0)),
            pl.BlockSpec((128, 128), lambda h, j: (0, 0)),
        ],
        out_specs=pl.BlockSpec((1, D // 8, 1, 8, 128), lambda h, j: (h, 0, j, 0, 0)),
        out_shape=jax.ShapeDtypeStruct((H, D // 8, nblk, 8, 128), jnp.float32),
    )


@functools.lru_cache(maxsize=None)
def _make_gather(B, D, C):
    b_per_w = B // _NW
    n_chunks = b_per_w // C
    n_groups = n_chunks // _NBUF
    mesh = plsc.VectorSubcoreMesh(core_axis_name="c", subcore_axis_name="s")

    scratch = (
        [pltpu.VMEM((C,), jnp.int32) for _ in range(_NBUF)]
        + [pltpu.VMEM((C, 2 * D), jnp.float32) for _ in range(_NBUF)]
        + [pltpu.SemaphoreType.DMA for _ in range(3 * _NBUF)]
    )

    @functools.partial(
        pl.kernel,
        mesh=mesh,
        out_type=jax.ShapeDtypeStruct((B, 2 * D), jnp.float32),
        scratch_types=scratch,
        compiler_params=pltpu.CompilerParams(use_tc_tiling_on_sc=False),
    )
    def gather_kernel(idx_hbm, table_hbm, out_hbm, *bufs):
        idx_v = bufs[:_NBUF]
        rows_v = bufs[_NBUF : 2 * _NBUF]
        sem_i = bufs[2 * _NBUF : 3 * _NBUF]
        sem_g = bufs[3 * _NBUF : 4 * _NBUF]
        sem_o = bufs[4 * _NBUF : 5 * _NBUF]

        wid = lax.axis_index("s") * _NC + lax.axis_index("c")
        base = wid * b_per_w

        for b in range(_NBUF):
            pltpu.async_copy(idx_hbm.at[pl.ds(base + b * C, C)], idx_v[b], sem_i[b])

        def body(g, carry):
            for b in range(_NBUF):
                chunk = g * _NBUF + b
                off = base + chunk * C
                pltpu.make_async_copy(
                    idx_hbm.at[pl.ds(off, C)], idx_v[b], sem_i[b]
                ).wait()

                @pl.when(g > 0)
                def _wait_prev_writeback(b=b, off=off):
                    pltpu.make_async_copy(
                        rows_v[b].at[:, pl.ds(0, D)],
                        out_hbm.at[pl.ds(off - _NBUF * C, C), pl.ds(0, D)],
                        sem_o[b],
                    ).wait()

                pltpu.async_copy(table_hbm.at[idx_v[b]], rows_v[b], sem_g[b])

            for b in range(_NBUF):
                chunk = g * _NBUF + b
                off = base + chunk * C
                pltpu.make_async_copy(
                    table_hbm.at[idx_v[b]], rows_v[b], sem_g[b]
                ).wait()

                @pl.when(chunk + _NBUF < n_chunks)
                def _prefetch_idx(b=b, off=off):
                    pltpu.async_copy(
                        idx_hbm.at[pl.ds(off + _NBUF * C, C)], idx_v[b], sem_i[b]
                    )

                pltpu.async_copy(
                    rows_v[b].at[:, pl.ds(0, D)], out_hbm.at[pl.ds(off, C), pl.ds(0, D)], sem_o[b]
                )
            return carry

        lax.fori_loop(0, n_groups, body, 0)

        last = base + (n_chunks - _NBUF) * C
        for b in range(_NBUF):
            pltpu.make_async_copy(
                rows_v[b].at[:, pl.ds(0, D)],
                out_hbm.at[pl.ds(last + b * C, C), pl.ds(0, D)],
                sem_o[b],
            ).wait()

    return gather_kernel


def kernel(x, embedding):
    batch, hist = x.shape
    vocab, dim = embedding.shape
    table = _make_pack(vocab, dim)(embedding.T, jnp.eye(dim, dtype=jnp.float32))
    flat_idx = x.T.reshape(batch * hist)
    g = _make_gather(batch * hist, dim, 400)(flat_idx, table)
    out5 = _make_tile(batch, hist, dim)(
        g.reshape(hist, batch, 2 * dim), jnp.eye(128, dtype=jnp.float32)
    )
    return out5.transpose(2, 4, 0, 1, 3).reshape(batch, hist, dim)


# speedup vs baseline: 4.1464x; 4.1464x over previous
import functools

import jax
import jax.numpy as jnp
from jax import lax
from jax.experimental import pallas as pl
from jax.experimental.pallas import tpu as pltpu
from jax.experimental.pallas import tpu_sc as plsc

_INFO = plsc.get_sparse_core_info()
_NC, _NS = _INFO.num_cores, _INFO.num_subcores
_NW = _NC * _NS

_NBUF = 2


@functools.lru_cache(maxsize=None)
def _make_gather(B, D, C):
    b_per_w = B // _NW
    n_chunks = b_per_w // C
    n_groups = n_chunks // _NBUF
    mesh = plsc.VectorSubcoreMesh(core_axis_name="c", subcore_axis_name="s")

    scratch = (
        [pltpu.VMEM((C,), jnp.int32) for _ in range(_NBUF)]
        + [pltpu.VMEM((C, D), jnp.float32) for _ in range(_NBUF)]
        + [pltpu.SemaphoreType.DMA for _ in range(3 * _NBUF)]
    )

    @functools.partial(
        pl.kernel,
        mesh=mesh,
        out_type=jax.ShapeDtypeStruct((B, D), jnp.float32),
        scratch_types=scratch,
        compiler_params=pltpu.CompilerParams(use_tc_tiling_on_sc=False),
    )
    def gather_kernel(idx_hbm, table_hbm, out_hbm, *bufs):
        idx_v = bufs[:_NBUF]
        rows_v = bufs[_NBUF : 2 * _NBUF]
        sem_i = bufs[2 * _NBUF : 3 * _NBUF]
        sem_g = bufs[3 * _NBUF : 4 * _NBUF]
        sem_o = bufs[4 * _NBUF : 5 * _NBUF]

        wid = lax.axis_index("s") * _NC + lax.axis_index("c")
        base = wid * b_per_w

        for b in range(_NBUF):
            pltpu.async_copy(idx_hbm.at[pl.ds(base + b * C, C)], idx_v[b], sem_i[b])

        def body(g, carry):
            for b in range(_NBUF):
                chunk = g * _NBUF + b
                off = base + chunk * C
                pltpu.make_async_copy(
                    idx_hbm.at[pl.ds(off, C)], idx_v[b], sem_i[b]
                ).wait()

                @pl.when(g > 0)
                def _wait_prev_writeback(b=b, off=off):
                    pltpu.make_async_copy(
                        rows_v[b], out_hbm.at[pl.ds(off - _NBUF * C, C)], sem_o[b]
                    ).wait()

                pltpu.async_copy(table_hbm.at[idx_v[b]], rows_v[b], sem_g[b])

            for b in range(_NBUF):
                chunk = g * _NBUF + b
                off = base + chunk * C
                pltpu.make_async_copy(
                    table_hbm.at[idx_v[b]], rows_v[b], sem_g[b]
                ).wait()

                @pl.when(chunk + _NBUF < n_chunks)
                def _prefetch_idx(b=b, off=off):
                    pltpu.async_copy(
                        idx_hbm.at[pl.ds(off + _NBUF * C, C)], idx_v[b], sem_i[b]
                    )

                pltpu.async_copy(rows_v[b], out_hbm.at[pl.ds(off, C)], sem_o[b])
            return carry

        lax.fori_loop(0, n_groups, body, 0)

        last = base + (n_chunks - _NBUF) * C
        for b in range(_NBUF):
            pltpu.make_async_copy(
                rows_v[b], out_hbm.at[pl.ds(last + b * C, C)], sem_o[b]
            ).wait()

    return gather_kernel


def kernel(x, embedding):
    batch, hist = x.shape
    dim = embedding.shape[1]
    flat_idx = x.reshape(batch * hist)
    out = _make_gather(batch * hist, dim, 800)(flat_idx, embedding)
    return out.reshape(batch, hist, dim)
